# baseline (speedup 1.0000x reference)
import functools

import jax
import jax.numpy as jnp
from jax import lax
from jax.experimental import pallas as pl
from jax.experimental.pallas import tpu as pltpu
from jax.experimental.pallas import tpu_sc as plsc

NC = 2
NT = 16
LANES = 16
CHUNK = 128
NBUF = 6
MDEPTH = 8
RC = 128
NB = 1000


def _scale_chunk(rows_ref, wbuf, rpar, mpar):
    ng = rows_ref.shape[2] // LANES

    @plsc.parallel_loop(0, CHUNK // LANES, unroll=2)
    def body(e0):
        w16 = wbuf[mpar, pl.ds(e0 * LANES, LANES)]
        for ee in range(LANES):
            wv = w16.at[jnp.full((LANES,), ee, dtype=jnp.int32)].get(
                mode="promise_in_bounds")
            e = e0 * LANES + ee
            for g in range(ng):
                sl = pl.ds(g * LANES, LANES)
                rows_ref[rpar, e, sl] = rows_ref[rpar, e, sl] * wv


def _sc_cheb(x_t, meta, w_r, *, n, fh, nch, n_rounds):
    b = x_t.shape[0]
    rows_pt = n // NT
    rc = RC if rows_pt % RC == 0 else rows_pt
    assert rows_pt % rc == 0
    n_rc = rows_pt // rc

    mesh = plsc.VectorSubcoreMesh(core_axis_name="c", subcore_axis_name="s")

    @functools.partial(
        pl.kernel,
        out_type=jax.ShapeDtypeStruct((n_rounds, b, NC, n, fh), jnp.float32),
        mesh=mesh,
        compiler_params=pltpu.CompilerParams(use_tc_tiling_on_sc=False),
        scratch_types=[
            pltpu.VMEM_SHARED((n, fh), jnp.float32),
            pltpu.VMEM((MDEPTH, 2, CHUNK), jnp.int32),
            pltpu.VMEM((MDEPTH, CHUNK), jnp.float32),
            pltpu.VMEM((NBUF, CHUNK, fh), jnp.float32),
            pltpu.VMEM((rc, fh), jnp.float32),
            pltpu.VMEM((rc, fh), jnp.float32),
            pltpu.VMEM((rc, fh), jnp.float32),
            pltpu.SemaphoreType.DMA((MDEPTH,)),
            pltpu.SemaphoreType.DMA((NBUF,)),
            pltpu.SemaphoreType.DMA((NBUF,)),
        ],
    )
    def cheb(x_hbm, meta_hbm, w_hbm, out_hbm,
             bufA, mbuf, wbuf, rows_v, ebuf, ebuf2, zbuf,
             msem, gsem, ssem):
        c = lax.axis_index("c")
        t = lax.axis_index("s")
        r0_tile = t * rows_pt
        ng = fh // LANES

        def zb(m, _):
            for g in range(ng):
                zbuf[m, pl.ds(g * LANES, LANES)] = jnp.zeros((LANES,), jnp.float32)
            return 0
        lax.fori_loop(0, rc, zb, 0)

        def start_meta(bb, j, mpar):
            pltpu.async_copy(meta_hbm.at[bb, t, j], mbuf.at[mpar], msem.at[mpar])
            pltpu.async_copy(w_hbm.at[bb, t, j], wbuf.at[mpar], msem.at[mpar])

        def wait_meta(bb, j, mpar):
            pltpu.make_async_copy(meta_hbm.at[bb, t, j], mbuf.at[mpar],
                                  msem.at[mpar]).wait()
            pltpu.make_async_copy(w_hbm.at[bb, t, j], wbuf.at[mpar],
                                  msem.at[mpar]).wait()

        def start_gather(src, mpar, rpar):
            pltpu.async_copy(src.at[mbuf.at[mpar, 0]], rows_v.at[rpar],
                             gsem.at[rpar])

        def wait_gather(src, mpar, rpar):
            pltpu.make_async_copy(src.at[mbuf.at[mpar, 0]], rows_v.at[rpar],
                                  gsem.at[rpar]).wait()

        def start_scatter(dst, mpar, rpar):
            pltpu.async_copy(rows_v.at[rpar], dst.at[mbuf.at[mpar, 1]],
                             ssem.at[rpar], add=True)

        def wait_scatter(dst, mpar, rpar):
            pltpu.make_async_copy(rows_v.at[rpar], dst.at[mbuf.at[mpar, 1]],
                                  ssem.at[rpar]).wait()

        def spmm_round(bb, src, dst):
            start_meta(bb, 0, 0)
            start_meta(bb, 1, 1)
            wait_meta(bb, 0, 0)
            start_gather(src, 0, 0)

            def body(j, _):
                @pl.when(j >= NBUF - 1)
                def _():
                    wait_scatter(dst, (j - NBUF + 1) % MDEPTH,
                                 (j - NBUF + 1) % NBUF)

                @pl.when(j + 2 < nch)
                def _():
                    start_meta(bb, j + 2, (j + 2) % MDEPTH)

                @pl.when(j + 1 < nch)
                def _():
                    wait_meta(bb, j + 1, (j + 1) % MDEPTH)
                    start_gather(src, (j + 1) % MDEPTH, (j + 1) % NBUF)

                wait_gather(src, j % MDEPTH, j % NBUF)
                _scale_chunk(rows_v, wbuf, j % NBUF, j % MDEPTH)
                start_scatter(dst, j % MDEPTH, j % NBUF)
                return 0
            lax.fori_loop(0, nch, body, 0)

            for j in range(nch - NBUF + 1, nch):
                wait_scatter(dst, j % MDEPTH, j % NBUF)

        def zero(dst):
            def zbody(i, _):
                pltpu.sync_copy(zbuf, dst.at[pl.ds(r0_tile + i * rc, rc)])
                return 0
            lax.fori_loop(0, n_rc, zbody, 0)

        def export(acc, prevprev_hbm, r, bb):
            def ebody(i, _):
                r0 = r0_tile + i * rc
                pltpu.sync_copy(acc.at[pl.ds(r0, rc)], ebuf)
                if prevprev_hbm is not None:
                    pltpu.sync_copy(prevprev_hbm.at[pl.ds(r0, rc)], ebuf2)

                    def comb(m, _):
                        for g in range(ng):
                            sl = pl.ds(g * LANES, LANES)
                            ebuf[m, sl] = ebuf[m, sl] * 2.0 - ebuf2[m, sl]
                        return 0
                    lax.fori_loop(0, rc, comb, 0)
                pltpu.sync_copy(ebuf, out_hbm.at[r - 1, bb, c, pl.ds(r0, rc)])
                return 0
            lax.fori_loop(0, n_rc, ebody, 0)

        for bb in range(b):
            zero(bufA)
            plsc.subcore_barrier()
            spmm_round(bb, x_hbm.at[bb, c], bufA)
            plsc.subcore_barrier()
            export(bufA, None, 1, bb)
            if n_rounds >= 2:
                zero(bufA)
                plsc.subcore_barrier()
                spmm_round(bb, out_hbm.at[0, bb, c], bufA)
                plsc.subcore_barrier()
                export(bufA, x_hbm.at[bb, c], 2, bb)
            if n_rounds >= 3:
                zero(bufA)
                plsc.subcore_barrier()
                spmm_round(bb, out_hbm.at[1, bb, c], bufA)
                plsc.subcore_barrier()
                export(bufA, out_hbm.at[0, bb, c], 3, bb)
            plsc.subcore_barrier()

    return cheb(x_t, meta, w_r)


def _combine_body(x, xs, w, b, out):
    km1 = xs.shape[0]
    fh = x.shape[-1]
    for o in range(out.shape[1]):
        for c in range(NC):
            acc = x[0, c] * w[0, o]
            for k in range(km1):
                acc = acc + xs[k, 0, c] * w[k + 1, o]
            out[0, o, :, pl.ds(c * fh, fh)] = jnp.maximum(acc + b[0, o, 0, 0], 0.0)


def _combine(x_t, xs, weight, bias, n, f):
    b = x_t.shape[0]
    fh = x_t.shape[-1]
    km1 = xs.shape[0]
    fo = weight.shape[1]
    grid = (b, n // NB)
    return pl.pallas_call(
        _combine_body,
        grid=grid,
        in_specs=[pl.BlockSpec((1, NC, NB, fh), lambda i, j: (i, 0, j, 0)),
                  pl.BlockSpec((km1, 1, NC, NB, fh), lambda i, j: (0, i, 0, j, 0)),
                  pl.BlockSpec(weight.shape, lambda i, j: (0, 0)),
                  pl.BlockSpec(bias.shape, lambda i, j: (0, 0, 0, 0))],
        out_specs=pl.BlockSpec((1, fo, NB, f), lambda i, j: (i, 0, j, 0)),
        out_shape=jax.ShapeDtypeStruct((b, fo, n, f), jnp.float32),
    )(x_t, xs, weight, bias)


def kernel(x, edge_index, edge_weight, weight, bias):
    b, n, f = x.shape
    e = edge_index.shape[2]
    k_deg = weight.shape[0]
    fh = f // NC

    ei = edge_index.astype(jnp.int32)
    row, col, w = ei[:, 0, :], ei[:, 1, :], edge_weight

    quant = NT * CHUNK * NBUF
    e_pad = ((e + quant - 1) // quant) * quant
    pad = e_pad - e
    if pad:
        row = jnp.pad(row, ((0, 0), (0, pad)))
        col = jnp.pad(col, ((0, 0), (0, pad)))
        w = jnp.pad(w, ((0, 0), (0, pad)))
    nch = e_pad // (NT * CHUNK)
    meta = jnp.stack(
        [col.reshape(b, NT, nch, CHUNK),
         row.reshape(b, NT, nch, CHUNK)], axis=3)
    w_r = w.reshape(b, NT, nch, CHUNK)

    n_quant = NT * RC
    n_pad = ((n + n_quant - 1) // n_quant) * n_quant
    x_t = x.reshape(b, n, NC, fh).transpose(0, 2, 1, 3)
    if n_pad != n:
        x_t = jnp.pad(x_t, ((0, 0), (0, 0), (0, n_pad - n), (0, 0)))

    xs = _sc_cheb(x_t, meta, w_r, n=n_pad, fh=fh, nch=nch,
                  n_rounds=k_deg - 1)
    return _combine(x_t, xs, weight, bias, n, f)


# speedup vs baseline: 2.1266x; 2.1266x over previous
import functools

import jax
import jax.numpy as jnp
from jax import lax
from jax.experimental import pallas as pl
from jax.experimental.pallas import tpu as pltpu
from jax.experimental.pallas import tpu_sc as plsc

NC = 2
NT = 16
LANES = 16
CHUNK = 128
NBUF = 4
MDEPTH = 8
RC = 64
NB = 1000


def _scale_chunk(rows_ref, wbuf, rpar, mpar):
    ng = rows_ref.shape[2] // LANES

    @plsc.parallel_loop(0, CHUNK // LANES, unroll=4)
    def body(e0):
        w16 = wbuf[mpar, pl.ds(e0 * LANES, LANES)]
        for ee in range(LANES):
            wv = w16.at[jnp.full((LANES,), ee, dtype=jnp.int32)].get(
                mode="promise_in_bounds")
            e = e0 * LANES + ee
            for g in range(ng):
                sl = pl.ds(g * LANES, LANES)
                rows_ref[rpar, e, sl] = rows_ref[rpar, e, sl] * wv


def _sc_cheb(x_t, meta, w_r, *, n, fh, nch, n_rounds):
    b = x_t.shape[0]
    rows_pt = n // NT
    rc = RC if rows_pt % RC == 0 else rows_pt
    assert rows_pt % rc == 0
    n_rc = rows_pt // rc

    mesh = plsc.VectorSubcoreMesh(core_axis_name="c", subcore_axis_name="s")

    @functools.partial(
        pl.kernel,
        out_type=jax.ShapeDtypeStruct((n_rounds, b, NC, n, fh), jnp.float32),
        mesh=mesh,
        compiler_params=pltpu.CompilerParams(use_tc_tiling_on_sc=False),
        scratch_types=[
            pltpu.VMEM_SHARED((n, fh), jnp.float32),
            pltpu.VMEM_SHARED((n, fh), jnp.float32),
            pltpu.VMEM((MDEPTH, 2, CHUNK), jnp.int32),
            pltpu.VMEM((MDEPTH, CHUNK), jnp.float32),
            pltpu.VMEM((NBUF, CHUNK, fh), jnp.float32),
            pltpu.VMEM((rc, fh), jnp.float32),
            pltpu.VMEM((rc, fh), jnp.float32),
            pltpu.VMEM((rc, fh), jnp.float32),
            pltpu.SemaphoreType.DMA((MDEPTH,)),
            pltpu.SemaphoreType.DMA((NBUF,)),
            pltpu.SemaphoreType.DMA((NBUF,)),
        ],
    )
    def cheb(x_hbm, meta_hbm, w_hbm, out_hbm,
             bufP, bufA, mbuf, wbuf, rows_v, ebuf, ebuf2, zbuf,
             msem, gsem, ssem):
        c = lax.axis_index("c")
        t = lax.axis_index("s")
        r0_tile = t * rows_pt
        ng = fh // LANES

        def zb(m, _):
            for g in range(ng):
                zbuf[m, pl.ds(g * LANES, LANES)] = jnp.zeros((LANES,), jnp.float32)
            return 0
        lax.fori_loop(0, rc, zb, 0)

        def start_meta(bb, j, mpar):
            pltpu.async_copy(meta_hbm.at[bb, t, j], mbuf.at[mpar], msem.at[mpar])
            pltpu.async_copy(w_hbm.at[bb, t, j], wbuf.at[mpar], msem.at[mpar])

        def wait_meta(bb, j, mpar):
            pltpu.make_async_copy(meta_hbm.at[bb, t, j], mbuf.at[mpar],
                                  msem.at[mpar]).wait()
            pltpu.make_async_copy(w_hbm.at[bb, t, j], wbuf.at[mpar],
                                  msem.at[mpar]).wait()

        def start_gather(src, mpar, rpar):
            pltpu.async_copy(src.at[mbuf.at[mpar, 0]], rows_v.at[rpar],
                             gsem.at[rpar])

        def wait_gather(src, mpar, rpar):
            pltpu.make_async_copy(src.at[mbuf.at[mpar, 0]], rows_v.at[rpar],
                                  gsem.at[rpar]).wait()

        def start_scatter(dst, mpar, rpar):
            pltpu.async_copy(rows_v.at[rpar], dst.at[mbuf.at[mpar, 1]],
                             ssem.at[rpar], add=True)

        def wait_scatter(dst, mpar, rpar):
            pltpu.make_async_copy(rows_v.at[rpar], dst.at[mbuf.at[mpar, 1]],
                                  ssem.at[rpar]).wait()

        def spmm_round(bb, src, dst):
            start_meta(bb, 0, 0)
            start_meta(bb, 1, 1)
            wait_meta(bb, 0, 0)
            start_gather(src, 0, 0)

            def body(j, _):
                @pl.when(j >= NBUF - 1)
                def _():
                    wait_scatter(dst, (j - NBUF + 1) % MDEPTH,
                                 (j - NBUF + 1) % NBUF)

                @pl.when(j + 2 < nch)
                def _():
                    start_meta(bb, j + 2, (j + 2) % MDEPTH)

                @pl.when(j + 1 < nch)
                def _():
                    wait_meta(bb, j + 1, (j + 1) % MDEPTH)
                    start_gather(src, (j + 1) % MDEPTH, (j + 1) % NBUF)

                wait_gather(src, j % MDEPTH, j % NBUF)
                _scale_chunk(rows_v, wbuf, j % NBUF, j % MDEPTH)
                start_scatter(dst, j % MDEPTH, j % NBUF)
                return 0
            lax.fori_loop(0, nch, body, 0)

            for j in range(nch - NBUF + 1, nch):
                wait_scatter(dst, j % MDEPTH, j % NBUF)

        def zero(dst):
            def zbody(i, _):
                pltpu.sync_copy(zbuf, dst.at[pl.ds(r0_tile + i * rc, rc)])
                return 0
            lax.fori_loop(0, n_rc, zbody, 0)

        def export(acc, prevprev_hbm, rewrite, r, bb):
            def ebody(i, _):
                r0 = r0_tile + i * rc
                pltpu.sync_copy(acc.at[pl.ds(r0, rc)], ebuf)
                if prevprev_hbm is not None:
                    pltpu.sync_copy(prevprev_hbm.at[pl.ds(r0, rc)], ebuf2)

                    def comb(m, _):
                        for g in range(ng):
                            sl = pl.ds(g * LANES, LANES)
                            ebuf[m, sl] = ebuf[m, sl] * 2.0 - ebuf2[m, sl]
                        return 0
                    lax.fori_loop(0, rc, comb, 0)
                    if rewrite:
                        pltpu.sync_copy(ebuf, acc.at[pl.ds(r0, rc)])
                pltpu.sync_copy(ebuf, out_hbm.at[r - 1, bb, c, pl.ds(r0, rc)])
                return 0
            lax.fori_loop(0, n_rc, ebody, 0)

        for bb in range(b):
            def sbody(i, _):
                r0 = r0_tile + i * rc
                pltpu.sync_copy(x_hbm.at[bb, c, pl.ds(r0, rc)], ebuf)
                pltpu.sync_copy(ebuf, bufP.at[pl.ds(r0, rc)])
                return 0
            lax.fori_loop(0, n_rc, sbody, 0)
            zero(bufA)
            plsc.subcore_barrier()

            spmm_round(bb, bufP, bufA)
            plsc.subcore_barrier()
            export(bufA, None, False, 1, bb)
            plsc.subcore_barrier()
            if n_rounds >= 2:
                zero(bufP)
                plsc.subcore_barrier()
                spmm_round(bb, bufA, bufP)
                plsc.subcore_barrier()
                export(bufP, x_hbm.at[bb, c], True, 2, bb)
                plsc.subcore_barrier()
            if n_rounds >= 3:
                zero(bufA)
                plsc.subcore_barrier()
                spmm_round(bb, bufP, bufA)
                plsc.subcore_barrier()
                export(bufA, out_hbm.at[0, bb, c], False, 3, bb)
                plsc.subcore_barrier()

    return cheb(x_t, meta, w_r)


def _combine_body(x, xs, w, b, out):
    km1 = xs.shape[0]
    fh = x.shape[-1]
    for o in range(out.shape[1]):
        for c in range(NC):
            acc = x[0, c] * w[0, o]
            for k in range(km1):
                acc = acc + xs[k, 0, c] * w[k + 1, o]
            out[0, o, :, pl.ds(c * fh, fh)] = jnp.maximum(acc + b[0, o, 0, 0], 0.0)


def _combine(x_t, xs, weight, bias, n, f):
    b = x_t.shape[0]
    fh = x_t.shape[-1]
    km1 = xs.shape[0]
    fo = weight.shape[1]
    grid = (b, n // NB)
    return pl.pallas_call(
        _combine_body,
        grid=grid,
        in_specs=[pl.BlockSpec((1, NC, NB, fh), lambda i, j: (i, 0, j, 0)),
                  pl.BlockSpec((km1, 1, NC, NB, fh), lambda i, j: (0, i, 0, j, 0)),
                  pl.BlockSpec(weight.shape, lambda i, j: (0, 0)),
                  pl.BlockSpec(bias.shape, lambda i, j: (0, 0, 0, 0))],
        out_specs=pl.BlockSpec((1, fo, NB, f), lambda i, j: (i, 0, j, 0)),
        out_shape=jax.ShapeDtypeStruct((b, fo, n, f), jnp.float32),
    )(x_t, xs, weight, bias)


def kernel(x, edge_index, edge_weight, weight, bias):
    b, n, f = x.shape
    e = edge_index.shape[2]
    k_deg = weight.shape[0]
    fh = f // NC

    ei = edge_index.astype(jnp.int32)
    row, col, w = ei[:, 0, :], ei[:, 1, :], edge_weight

    quant = NT * CHUNK * NBUF
    e_pad = ((e + quant - 1) // quant) * quant
    pad = e_pad - e
    if pad:
        row = jnp.pad(row, ((0, 0), (0, pad)))
        col = jnp.pad(col, ((0, 0), (0, pad)))
        w = jnp.pad(w, ((0, 0), (0, pad)))
    nch = e_pad // (NT * CHUNK)
    meta = jnp.stack(
        [col.reshape(b, NT, nch, CHUNK),
         row.reshape(b, NT, nch, CHUNK)], axis=3)
    w_r = w.reshape(b, NT, nch, CHUNK)

    n_quant = NT * RC
    n_pad = ((n + n_quant - 1) // n_quant) * n_quant
    x_t = x.reshape(b, n, NC, fh).transpose(0, 2, 1, 3)
    if n_pad != n:
        x_t = jnp.pad(x_t, ((0, 0), (0, 0), (0, n_pad - n), (0, 0)))

    xs = _sc_cheb(x_t, meta, w_r, n=n_pad, fh=fh, nch=nch,
                  n_rounds=k_deg - 1)
    return _combine(x_t, xs, weight, bias, n, f)


# speedup vs baseline: 2.2159x; 1.0420x over previous
import functools

import jax
import jax.numpy as jnp
from jax import lax
from jax.experimental import pallas as pl
from jax.experimental.pallas import tpu as pltpu
from jax.experimental.pallas import tpu_sc as plsc

NC = 2
NT = 16
LANES = 16
CHUNK = 128
NBUF = 4
MDEPTH = 8
RC = 64
NB = 1000


def _scale_chunk(rows_ref, wbuf, rpar, mpar):
    ng = rows_ref.shape[2] // LANES

    @plsc.parallel_loop(0, CHUNK // LANES, unroll=2)
    def body(e0):
        w16 = wbuf[mpar, pl.ds(e0 * LANES, LANES)]
        for ee in range(LANES):
            wv = w16.at[jnp.full((LANES,), ee, dtype=jnp.int32)].get(
                mode="promise_in_bounds")
            e = e0 * LANES + ee
            for g in range(ng):
                sl = pl.ds(g * LANES, LANES)
                rows_ref[rpar, e, sl] = rows_ref[rpar, e, sl] * wv


def _sc_cheb(x_t, meta, w_r, *, n, fh, nch, n_rounds):
    b = x_t.shape[0]
    rows_pt = n // NT
    rc = RC if rows_pt % RC == 0 else rows_pt
    assert rows_pt % rc == 0
    n_rc = rows_pt // rc

    mesh = plsc.VectorSubcoreMesh(core_axis_name="c", subcore_axis_name="s")

    @functools.partial(
        pl.kernel,
        out_type=jax.ShapeDtypeStruct((n_rounds, b, NC, n, fh), jnp.float32),
        mesh=mesh,
        compiler_params=pltpu.CompilerParams(use_tc_tiling_on_sc=False),
        scratch_types=[
            pltpu.VMEM_SHARED((n, fh), jnp.float32),
            pltpu.VMEM_SHARED((n, fh), jnp.float32),
            pltpu.VMEM((MDEPTH, 2, CHUNK), jnp.int32),
            pltpu.VMEM((MDEPTH, CHUNK), jnp.float32),
            pltpu.VMEM((NBUF, CHUNK, fh), jnp.float32),
            pltpu.VMEM((rc, fh), jnp.float32),
            pltpu.VMEM((rc, fh), jnp.float32),
            pltpu.VMEM((rc, fh), jnp.float32),
            pltpu.SemaphoreType.DMA((MDEPTH,)),
            pltpu.SemaphoreType.DMA((NBUF,)),
            pltpu.SemaphoreType.DMA((NBUF,)),
        ],
    )
    def cheb(x_hbm, meta_hbm, w_hbm, out_hbm,
             bufP, bufA, mbuf, wbuf, rows_v, ebuf, ebuf2, zbuf,
             msem, gsem, ssem):
        c = lax.axis_index("c")
        t = lax.axis_index("s")
        r0_tile = t * rows_pt
        ng = fh // LANES

        def zb(m, _):
            for g in range(ng):
                zbuf[m, pl.ds(g * LANES, LANES)] = jnp.zeros((LANES,), jnp.float32)
            return 0
        lax.fori_loop(0, rc, zb, 0)

        def start_meta(bb, j, mpar):
            pltpu.async_copy(meta_hbm.at[bb, t, j], mbuf.at[mpar], msem.at[mpar])
            pltpu.async_copy(w_hbm.at[bb, t, j], wbuf.at[mpar], msem.at[mpar])

        def wait_meta(bb, j, mpar):
            pltpu.make_async_copy(meta_hbm.at[bb, t, j], mbuf.at[mpar],
                                  msem.at[mpar]).wait()
            pltpu.make_async_copy(w_hbm.at[bb, t, j], wbuf.at[mpar],
                                  msem.at[mpar]).wait()

        def start_gather(src, mpar, rpar):
            pltpu.async_copy(src.at[mbuf.at[mpar, 0]], rows_v.at[rpar],
                             gsem.at[rpar])

        def wait_gather(src, mpar, rpar):
            pltpu.make_async_copy(src.at[mbuf.at[mpar, 0]], rows_v.at[rpar],
                                  gsem.at[rpar]).wait()

        def start_scatter(dst, mpar, rpar):
            pltpu.async_copy(rows_v.at[rpar], dst.at[mbuf.at[mpar, 1]],
                             ssem.at[rpar], add=True)

        def wait_scatter(dst, mpar, rpar):
            pltpu.make_async_copy(rows_v.at[rpar], dst.at[mbuf.at[mpar, 1]],
                                  ssem.at[rpar]).wait()

        def spmm_round(bb, src, dst):
            start_meta(bb, 0, 0)
            start_meta(bb, 1, 1)
            wait_meta(bb, 0, 0)
            start_gather(src, 0, 0)

            def body(j, _):
                @pl.when(j >= NBUF - 1)
                def _():
                    wait_scatter(dst, (j - NBUF + 1) % MDEPTH,
                                 (j - NBUF + 1) % NBUF)

                @pl.when(j + 2 < nch)
                def _():
                    start_meta(bb, j + 2, (j + 2) % MDEPTH)

                @pl.when(j + 1 < nch)
                def _():
                    wait_meta(bb, j + 1, (j + 1) % MDEPTH)
                    start_gather(src, (j + 1) % MDEPTH, (j + 1) % NBUF)

                wait_gather(src, j % MDEPTH, j % NBUF)
                _scale_chunk(rows_v, wbuf, j % NBUF, j % MDEPTH)
                start_scatter(dst, j % MDEPTH, j % NBUF)
                return 0
            lax.fori_loop(0, nch, body, 0)

            for j in range(nch - NBUF + 1, nch):
                wait_scatter(dst, j % MDEPTH, j % NBUF)

        def zero(dst):
            def zbody(i, _):
                pltpu.sync_copy(zbuf, dst.at[pl.ds(r0_tile + i * rc, rc)])
                return 0
            lax.fori_loop(0, n_rc, zbody, 0)

        def export(acc, prevprev_hbm, rewrite, r, bb):
            def ebody(i, _):
                r0 = r0_tile + i * rc
                pltpu.sync_copy(acc.at[pl.ds(r0, rc)], ebuf)
                if prevprev_hbm is not None:
                    pltpu.sync_copy(prevprev_hbm.at[pl.ds(r0, rc)], ebuf2)

                    def comb(m, _):
                        for g in range(ng):
                            sl = pl.ds(g * LANES, LANES)
                            ebuf[m, sl] = ebuf[m, sl] * 2.0 - ebuf2[m, sl]
                        return 0
                    lax.fori_loop(0, rc, comb, 0)
                    if rewrite:
                        pltpu.sync_copy(ebuf, acc.at[pl.ds(r0, rc)])
                pltpu.sync_copy(ebuf, out_hbm.at[r - 1, bb, c, pl.ds(r0, rc)])
                return 0
            lax.fori_loop(0, n_rc, ebody, 0)

        for bb in range(b):
            def sbody(i, _):
                r0 = r0_tile + i * rc
                pltpu.sync_copy(x_hbm.at[bb, c, pl.ds(r0, rc)], ebuf)
                pltpu.sync_copy(ebuf, bufP.at[pl.ds(r0, rc)])
                return 0
            lax.fori_loop(0, n_rc, sbody, 0)
            zero(bufA)
            plsc.subcore_barrier()

            spmm_round(bb, bufP, bufA)
            plsc.subcore_barrier()
            export(bufA, None, False, 1, bb)
            plsc.subcore_barrier()
            if n_rounds >= 2:
                zero(bufP)
                plsc.subcore_barrier()
                spmm_round(bb, bufA, bufP)
                plsc.subcore_barrier()
                export(bufP, x_hbm.at[bb, c], True, 2, bb)
                plsc.subcore_barrier()
            if n_rounds >= 3:
                zero(bufA)
                plsc.subcore_barrier()
                spmm_round(bb, bufP, bufA)
                plsc.subcore_barrier()
                export(bufA, out_hbm.at[0, bb, c], False, 3, bb)
                plsc.subcore_barrier()

    return cheb(x_t, meta, w_r)


def _combine_body(x, xs, w, b, out):
    km1 = xs.shape[0]
    fh = x.shape[-1]
    for o in range(out.shape[1]):
        for c in range(NC):
            acc = x[0, c] * w[0, o]
            for k in range(km1):
                acc = acc + xs[k, 0, c] * w[k + 1, o]
            out[0, o, :, pl.ds(c * fh, fh)] = jnp.maximum(acc + b[0, o, 0, 0], 0.0)


def _combine_batch_body(x, xs, w, b, out):
    _combine_body(x, xs, w, b, out)


def _combine_batch_body_aliased(x, xs, w, b, prev, out):
    del prev
    _combine_body(x, xs, w, b, out)


def _combine_batch(x_t, xs_b, weight, bias, bb, out_prev, out_shape):
    fh = x_t.shape[-1]
    km1 = xs_b.shape[0]
    _, fo, n, f = out_shape
    grid = (n // NB,)
    in_specs = [pl.BlockSpec((1, NC, NB, fh), lambda j: (bb, 0, j, 0)),
                pl.BlockSpec((km1, 1, NC, NB, fh), lambda j: (0, 0, 0, j, 0)),
                pl.BlockSpec(weight.shape, lambda j: (0, 0)),
                pl.BlockSpec(bias.shape, lambda j: (0, 0, 0, 0))]
    args = [x_t, xs_b, weight, bias]
    kwargs = {}
    body = _combine_batch_body
    if out_prev is not None:
        in_specs.append(pl.BlockSpec(memory_space=pltpu.MemorySpace.HBM))
        args.append(out_prev)
        kwargs["input_output_aliases"] = {4: 0}
        body = _combine_batch_body_aliased
    return pl.pallas_call(
        body,
        grid=grid,
        in_specs=in_specs,
        out_specs=pl.BlockSpec((1, fo, NB, f), lambda j: (bb, 0, j, 0)),
        out_shape=jax.ShapeDtypeStruct(out_shape, jnp.float32),
        **kwargs,
    )(*args)


def _combine(x_t, xs, weight, bias, n, f):
    b = x_t.shape[0]
    fh = x_t.shape[-1]
    km1 = xs.shape[0]
    fo = weight.shape[1]
    grid = (b, n // NB)
    return pl.pallas_call(
        _combine_body,
        grid=grid,
        in_specs=[pl.BlockSpec((1, NC, NB, fh), lambda i, j: (i, 0, j, 0)),
                  pl.BlockSpec((km1, 1, NC, NB, fh), lambda i, j: (0, i, 0, j, 0)),
                  pl.BlockSpec(weight.shape, lambda i, j: (0, 0)),
                  pl.BlockSpec(bias.shape, lambda i, j: (0, 0, 0, 0))],
        out_specs=pl.BlockSpec((1, fo, NB, f), lambda i, j: (i, 0, j, 0)),
        out_shape=jax.ShapeDtypeStruct((b, fo, n, f), jnp.float32),
    )(x_t, xs, weight, bias)


def kernel(x, edge_index, edge_weight, weight, bias):
    b, n, f = x.shape
    e = edge_index.shape[2]
    k_deg = weight.shape[0]
    fo = weight.shape[1]
    fh = f // NC

    ei = edge_index.astype(jnp.int32)
    row, col, w = ei[:, 0, :], ei[:, 1, :], edge_weight

    quant = NT * CHUNK * NBUF
    e_pad = ((e + quant - 1) // quant) * quant
    pad = e_pad - e
    if pad:
        row = jnp.pad(row, ((0, 0), (0, pad)))
        col = jnp.pad(col, ((0, 0), (0, pad)))
        w = jnp.pad(w, ((0, 0), (0, pad)))
    nch = e_pad // (NT * CHUNK)
    meta = jnp.stack(
        [col.reshape(b, NT, nch, CHUNK),
         row.reshape(b, NT, nch, CHUNK)], axis=3)
    w_r = w.reshape(b, NT, nch, CHUNK)

    n_quant = NT * RC
    n_pad = ((n + n_quant - 1) // n_quant) * n_quant
    x_t = x.reshape(b, n, NC, fh).transpose(0, 2, 1, 3)
    if n_pad != n:
        x_t = jnp.pad(x_t, ((0, 0), (0, 0), (0, n_pad - n), (0, 0)))

    if b == 2:
        xs0 = _sc_cheb(x_t[0:1], meta[0:1], w_r[0:1], n=n_pad, fh=fh,
                       nch=nch, n_rounds=k_deg - 1)
        xs1 = _sc_cheb(x_t[1:2], meta[1:2], w_r[1:2], n=n_pad, fh=fh,
                       nch=nch, n_rounds=k_deg - 1)
        o0 = _combine_batch(x_t, xs0, weight, bias, 0, None, (b, fo, n, f))
        return _combine_batch(x_t, xs1, weight, bias, 1, o0, (b, fo, n, f))
    xs = _sc_cheb(x_t, meta, w_r, n=n_pad, fh=fh, nch=nch,
                  n_rounds=k_deg - 1)
    return _combine(x_t, xs, weight, bias, n, f)


# speedup vs baseline: 2.3989x; 1.0826x over previous
import functools

import jax
import jax.numpy as jnp
from jax import lax
from jax.experimental import pallas as pl
from jax.experimental.pallas import tpu as pltpu
from jax.experimental.pallas import tpu_sc as plsc

NC = 2
NT = 16
LANES = 16
CHUNK = 128
NBUF = 5
MDEPTH = 8
RC = 128
NB = 1000


def _scale_chunk(rows_ref, wbuf, rpar, mpar):
    ng = rows_ref.shape[2] // LANES

    @plsc.parallel_loop(0, CHUNK // LANES, unroll=2)
    def body(e0):
        w16 = wbuf[mpar, pl.ds(e0 * LANES, LANES)]
        for ee in range(LANES):
            wv = w16.at[jnp.full((LANES,), ee, dtype=jnp.int32)].get(
                mode="promise_in_bounds")
            e = e0 * LANES + ee
            for g in range(ng):
                sl = pl.ds(g * LANES, LANES)
                rows_ref[rpar, e, sl] = rows_ref[rpar, e, sl] * wv


def _sc_cheb(x_t, meta, w_r, *, n, fh, nch, n_rounds):
    b = x_t.shape[0]
    rows_pt = n // NT
    rc = RC if rows_pt % RC == 0 else rows_pt
    assert rows_pt % rc == 0
    n_rc = rows_pt // rc

    mesh = plsc.VectorSubcoreMesh(core_axis_name="c", subcore_axis_name="s")

    @functools.partial(
        pl.kernel,
        out_type=jax.ShapeDtypeStruct((n_rounds, b, NC, n, fh), jnp.float32),
        mesh=mesh,
        compiler_params=pltpu.CompilerParams(use_tc_tiling_on_sc=False),
        scratch_types=[
            pltpu.VMEM_SHARED((n, fh), jnp.float32),
            pltpu.VMEM_SHARED((n, fh), jnp.float32),
            pltpu.VMEM((MDEPTH, 2, CHUNK), jnp.int32),
            pltpu.VMEM((MDEPTH, CHUNK), jnp.float32),
            pltpu.VMEM((NBUF, CHUNK, fh), jnp.float32),
            pltpu.SemaphoreType.DMA((MDEPTH,)),
            pltpu.SemaphoreType.DMA((NBUF,)),
            pltpu.SemaphoreType.DMA((NBUF,)),
        ],
    )
    def cheb(x_hbm, meta_hbm, w_hbm, out_hbm,
             bufP, bufA, mbuf, wbuf, rows_v,
             msem, gsem, ssem):
        c = lax.axis_index("c")
        t = lax.axis_index("s")
        r0_tile = t * rows_pt
        ng = fh // LANES
        ebuf = rows_v.at[0, pl.ds(0, rc)]
        ebuf2 = rows_v.at[1, pl.ds(0, rc)]
        zbuf = rows_v.at[2, pl.ds(0, rc)]

        def start_meta(bb, j, mpar):
            pltpu.async_copy(meta_hbm.at[bb, t, j], mbuf.at[mpar], msem.at[mpar])
            pltpu.async_copy(w_hbm.at[bb, t, j], wbuf.at[mpar], msem.at[mpar])

        def wait_meta(bb, j, mpar):
            pltpu.make_async_copy(meta_hbm.at[bb, t, j], mbuf.at[mpar],
                                  msem.at[mpar]).wait()
            pltpu.make_async_copy(w_hbm.at[bb, t, j], wbuf.at[mpar],
                                  msem.at[mpar]).wait()

        def start_gather(src, mpar, rpar):
            pltpu.async_copy(src.at[mbuf.at[mpar, 0]], rows_v.at[rpar],
                             gsem.at[rpar])

        def wait_gather(src, mpar, rpar):
            pltpu.make_async_copy(src.at[mbuf.at[mpar, 0]], rows_v.at[rpar],
                                  gsem.at[rpar]).wait()

        def start_scatter(dst, mpar, rpar):
            pltpu.async_copy(rows_v.at[rpar], dst.at[mbuf.at[mpar, 1]],
                             ssem.at[rpar], add=True)

        def wait_scatter(dst, mpar, rpar):
            pltpu.make_async_copy(rows_v.at[rpar], dst.at[mbuf.at[mpar, 1]],
                                  ssem.at[rpar]).wait()

        def spmm_round(bb, src, dst):
            start_meta(bb, 0, 0)
            start_meta(bb, 1, 1)
            start_meta(bb, 2, 2)
            wait_meta(bb, 0, 0)
            start_gather(src, 0, 0)
            wait_meta(bb, 1, 1)
            start_gather(src, 1, 1)

            def body(j, _):
                @pl.when(j >= NBUF - 2)
                def _():
                    wait_scatter(dst, (j - NBUF + 2) % MDEPTH,
                                 (j - NBUF + 2) % NBUF)

                @pl.when(j + 3 < nch)
                def _():
                    start_meta(bb, j + 3, (j + 3) % MDEPTH)

                @pl.when(j + 2 < nch)
                def _():
                    wait_meta(bb, j + 2, (j + 2) % MDEPTH)
                    start_gather(src, (j + 2) % MDEPTH, (j + 2) % NBUF)

                wait_gather(src, j % MDEPTH, j % NBUF)
                _scale_chunk(rows_v, wbuf, j % NBUF, j % MDEPTH)
                start_scatter(dst, j % MDEPTH, j % NBUF)
                return 0
            lax.fori_loop(0, nch, body, 0)

            for j in range(nch - NBUF + 2, nch):
                wait_scatter(dst, j % MDEPTH, j % NBUF)

        def zero(dst):
            def zb(m, _):
                for g in range(ng):
                    zbuf[m, pl.ds(g * LANES, LANES)] = jnp.zeros(
                        (LANES,), jnp.float32)
                return 0
            lax.fori_loop(0, rc, zb, 0)

            def zbody(i, _):
                pltpu.sync_copy(zbuf, dst.at[pl.ds(r0_tile + i * rc, rc)])
                return 0
            lax.fori_loop(0, n_rc, zbody, 0)

        def export(acc, prevprev_hbm, rewrite, r, bb):
            def ebody(i, _):
                r0 = r0_tile + i * rc
                pltpu.sync_copy(acc.at[pl.ds(r0, rc)], ebuf)
                if prevprev_hbm is not None:
                    pltpu.sync_copy(prevprev_hbm.at[pl.ds(r0, rc)], ebuf2)

                    def comb(m, _):
                        for g in range(ng):
                            sl = pl.ds(g * LANES, LANES)
                            ebuf[m, sl] = ebuf[m, sl] * 2.0 - ebuf2[m, sl]
                        return 0
                    lax.fori_loop(0, rc, comb, 0)
                    if rewrite:
                        pltpu.sync_copy(ebuf, acc.at[pl.ds(r0, rc)])
                pltpu.sync_copy(ebuf, out_hbm.at[r - 1, bb, c, pl.ds(r0, rc)])
                return 0
            lax.fori_loop(0, n_rc, ebody, 0)

        for bb in range(b):
            def sbody(i, _):
                r0 = r0_tile + i * rc
                pltpu.sync_copy(x_hbm.at[bb, c, pl.ds(r0, rc)], ebuf)
                pltpu.sync_copy(ebuf, bufP.at[pl.ds(r0, rc)])
                return 0
            lax.fori_loop(0, n_rc, sbody, 0)
            zero(bufA)
            plsc.subcore_barrier()

            spmm_round(bb, bufP, bufA)
            plsc.subcore_barrier()
            export(bufA, None, False, 1, bb)
            plsc.subcore_barrier()
            if n_rounds >= 2:
                zero(bufP)
                plsc.subcore_barrier()
                spmm_round(bb, bufA, bufP)
                plsc.subcore_barrier()
                export(bufP, x_hbm.at[bb, c], True, 2, bb)
                plsc.subcore_barrier()
            if n_rounds >= 3:
                zero(bufA)
                plsc.subcore_barrier()
                spmm_round(bb, bufP, bufA)
                plsc.subcore_barrier()
                export(bufA, out_hbm.at[0, bb, c], False, 3, bb)
                plsc.subcore_barrier()

    return cheb(x_t, meta, w_r)


def _combine_body(x, xs, w, b, out):
    km1 = xs.shape[0]
    fh = x.shape[-1]
    for o in range(out.shape[1]):
        for c in range(NC):
            acc = x[0, c] * w[0, o]
            for k in range(km1):
                acc = acc + xs[k, 0, c] * w[k + 1, o]
            out[0, o, :, pl.ds(c * fh, fh)] = jnp.maximum(acc + b[0, o, 0, 0], 0.0)


def _combine_batch_body(x, xs, w, b, out):
    _combine_body(x, xs, w, b, out)


def _combine_batch_body_aliased(x, xs, w, b, prev, out):
    del prev
    _combine_body(x, xs, w, b, out)


def _combine_batch(x_t, xs_b, weight, bias, bb, out_prev, out_shape):
    fh = x_t.shape[-1]
    km1 = xs_b.shape[0]
    _, fo, n, f = out_shape
    grid = (n // NB,)
    in_specs = [pl.BlockSpec((1, NC, NB, fh), lambda j: (bb, 0, j, 0)),
                pl.BlockSpec((km1, 1, NC, NB, fh), lambda j: (0, 0, 0, j, 0)),
                pl.BlockSpec(weight.shape, lambda j: (0, 0)),
                pl.BlockSpec(bias.shape, lambda j: (0, 0, 0, 0))]
    args = [x_t, xs_b, weight, bias]
    kwargs = {}
    body = _combine_batch_body
    if out_prev is not None:
        in_specs.append(pl.BlockSpec(memory_space=pltpu.MemorySpace.HBM))
        args.append(out_prev)
        kwargs["input_output_aliases"] = {4: 0}
        body = _combine_batch_body_aliased
    return pl.pallas_call(
        body,
        grid=grid,
        in_specs=in_specs,
        out_specs=pl.BlockSpec((1, fo, NB, f), lambda j: (bb, 0, j, 0)),
        out_shape=jax.ShapeDtypeStruct(out_shape, jnp.float32),
        **kwargs,
    )(*args)


def _combine(x_t, xs, weight, bias, n, f):
    b = x_t.shape[0]
    fh = x_t.shape[-1]
    km1 = xs.shape[0]
    fo = weight.shape[1]
    grid = (b, n // NB)
    return pl.pallas_call(
        _combine_body,
        grid=grid,
        in_specs=[pl.BlockSpec((1, NC, NB, fh), lambda i, j: (i, 0, j, 0)),
                  pl.BlockSpec((km1, 1, NC, NB, fh), lambda i, j: (0, i, 0, j, 0)),
                  pl.BlockSpec(weight.shape, lambda i, j: (0, 0)),
                  pl.BlockSpec(bias.shape, lambda i, j: (0, 0, 0, 0))],
        out_specs=pl.BlockSpec((1, fo, NB, f), lambda i, j: (i, 0, j, 0)),
        out_shape=jax.ShapeDtypeStruct((b, fo, n, f), jnp.float32),
    )(x_t, xs, weight, bias)


def kernel(x, edge_index, edge_weight, weight, bias):
    b, n, f = x.shape
    e = edge_index.shape[2]
    k_deg = weight.shape[0]
    fo = weight.shape[1]
    fh = f // NC

    ei = edge_index.astype(jnp.int32)
    row, col, w = ei[:, 0, :], ei[:, 1, :], edge_weight

    quant = NT * CHUNK * NBUF
    e_pad = ((e + quant - 1) // quant) * quant
    pad = e_pad - e
    if pad:
        row = jnp.pad(row, ((0, 0), (0, pad)))
        col = jnp.pad(col, ((0, 0), (0, pad)))
        w = jnp.pad(w, ((0, 0), (0, pad)))
    nch = e_pad // (NT * CHUNK)
    meta = jnp.stack(
        [col.reshape(b, NT, nch, CHUNK),
         row.reshape(b, NT, nch, CHUNK)], axis=3)
    w_r = w.reshape(b, NT, nch, CHUNK)

    n_quant = NT * RC
    n_pad = ((n + n_quant - 1) // n_quant) * n_quant
    x_t = x.reshape(b, n, NC, fh).transpose(0, 2, 1, 3)
    if n_pad != n:
        x_t = jnp.pad(x_t, ((0, 0), (0, 0), (0, n_pad - n), (0, 0)))

    if b == 2:
        xs0 = _sc_cheb(x_t[0:1], meta[0:1], w_r[0:1], n=n_pad, fh=fh,
                       nch=nch, n_rounds=k_deg - 1)
        xs1 = _sc_cheb(x_t[1:2], meta[1:2], w_r[1:2], n=n_pad, fh=fh,
                       nch=nch, n_rounds=k_deg - 1)
        o0 = _combine_batch(x_t, xs0, weight, bias, 0, None, (b, fo, n, f))
        return _combine_batch(x_t, xs1, weight, bias, 1, o0, (b, fo, n, f))
    xs = _sc_cheb(x_t, meta, w_r, n=n_pad, fh=fh, nch=nch,
                  n_rounds=k_deg - 1)
    return _combine(x_t, xs, weight, bias, n, f)
